# Initial kernel scaffold; baseline (speedup 1.0000x reference)
import jax
import jax.numpy as jnp
from jax.experimental import pallas as pl


def kernel(edge_attr, W0, W1, W2):
    raise NotImplementedError("write your pallas kernel here")


# speedup vs baseline: 1.0918x; 1.0918x over previous
import functools

import jax
import jax.numpy as jnp
from jax import lax
from jax.experimental import pallas as pl
from jax.experimental.pallas import tpu as pltpu
from jax.experimental.pallas import tpu_sc as plsc

D = 128
V0, V1, V2 = 5, 6, 2
VT = V0 * V1 * V2

NC, NS = 2, 16
NW = NC * NS

C = 80
KF = 5
CB = C * KF


def _table_body(w0_ref, w1_ref, w2_ref, t_ref):
    for r in range(VT):
        i0, i1, i2 = r // (V1 * V2), (r // V2) % V1, r % V2
        t_ref[pl.ds(r, 1), :] = (
            w0_ref[pl.ds(i0, 1), :]
            + w1_ref[pl.ds(i1, 1), :]
            + w2_ref[pl.ds(i2, 1), :]
        )


def _build_table(W0, W1, W2):
    return pl.pallas_call(
        _table_body,
        out_shape=jax.ShapeDtypeStruct((VT, D), jnp.float32),
    )(W0, W1, W2)


def _sc_body(bpw, tab_hbm, ea0_hbm, ea1_hbm, ea2_hbm, out_hbm, ea0_v, ea1_v,
             ea2_v, idx_v, rows_v, gsem):
    wid = lax.axis_index("s") * NC + lax.axis_index("c")
    base = wid * bpw
    pltpu.sync_copy(ea0_hbm.at[pl.ds(base, bpw)], ea0_v)
    pltpu.sync_copy(ea1_hbm.at[pl.ds(base, bpw)], ea1_v)
    pltpu.sync_copy(ea2_hbm.at[pl.ds(base, bpw)], ea2_v)

    def idx_body(j, carry):
        i0 = ea0_v[pl.ds(j * 16, 16)]
        i1 = ea1_v[pl.ds(j * 16, 16)]
        i2 = ea2_v[pl.ds(j * 16, 16)]
        cidx = i0 * (V1 * V2) + i1 * V2 + i2
        idx_v[j // (C // 16), pl.ds((j % (C // 16)) * 16, 16)] = cidx
        return carry

    lax.fori_loop(0, bpw // 16, idx_body, 0)

    n_outer = bpw // CB

    def gather_body(o, carry):
        cps = []
        for f in range(KF):
            k = o * KF + f
            cps.append(
                pltpu.async_copy(
                    tab_hbm.at[idx_v.at[k]],
                    rows_v.at[pl.ds(f * C, C)],
                    gsem,
                )
            )
        for cp in cps:
            cp.wait()
        pltpu.sync_copy(rows_v, out_hbm.at[pl.ds(base + o * CB, CB)])
        return carry

    lax.fori_loop(0, n_outer, gather_body, 0)


def kernel(edge_attr, W0, W1, W2):
    E = edge_attr.shape[0]
    assert E % (NW * CB) == 0
    bpw = E // NW

    table = _build_table(W0, W1, W2)
    ea = edge_attr.astype(jnp.int32)
    ea0, ea1, ea2 = ea[:, 0], ea[:, 1], ea[:, 2]

    mesh = plsc.VectorSubcoreMesh(core_axis_name="c", subcore_axis_name="s")
    sc_kernel = functools.partial(
        pl.kernel,
        out_type=jax.ShapeDtypeStruct((E, D), jnp.float32),
        mesh=mesh,
        scratch_types=[
            pltpu.VMEM((bpw,), jnp.int32),
            pltpu.VMEM((bpw,), jnp.int32),
            pltpu.VMEM((bpw,), jnp.int32),
            pltpu.VMEM((bpw // C, C), jnp.int32),
            pltpu.VMEM((CB, D), jnp.float32),
            pltpu.SemaphoreType.DMA,
        ],
    )(functools.partial(_sc_body, bpw))
    return sc_kernel(table, ea0, ea1, ea2)
